# Initial kernel scaffold; baseline (speedup 1.0000x reference)
import jax
import jax.numpy as jnp
from jax.experimental import pallas as pl


def kernel(x, edge_index, fc_W, fc_b, bn0_g, bn0_b, conv_W1, conv_b1, bn1_g, bn1_b, conv_W2, conv_b2, bn2_g, bn2_b):
    raise NotImplementedError("write your pallas kernel here")


# speedup vs baseline: 13.5715x; 13.5715x over previous
import functools

import jax
import jax.numpy as jnp
from jax import lax
from jax.experimental import pallas as pl
from jax.experimental.pallas import tpu as pltpu
from jax.experimental.pallas import tpu_sc as plsc

N_NODES = 10000
D = 128
EPS = 1e-5

NC = 2
NS = 16
L = 16
NW = NC * NS

CHUNK = 128
N_PAD = 10240
ROWS_PER_TILE = N_PAD // NS
DEG_W = 8


def _zero_vmem_rows(ref, nrows, ncols):
    z = jnp.zeros((L,), jnp.float32)

    def body(r, carry):
        for k in range(ncols // L):
            ref[r, pl.ds(k * L, L)] = z
        return carry

    lax.fori_loop(0, nrows, body, 0)


def _fill_ones_rows(ref, nrows, ncols):
    o = jnp.ones((L,), jnp.float32)

    def body(r, carry):
        for k in range(ncols // L):
            ref[r, pl.ds(k * L, L)] = o
        return carry

    lax.fori_loop(0, nrows, body, 0)


def _sc_degree(dst_idx):
    nchunks = dst_idx.shape[1]
    mesh = plsc.VectorSubcoreMesh(core_axis_name="c", subcore_axis_name="s")

    @functools.partial(
        pl.kernel,
        out_type=jax.ShapeDtypeStruct((NC, N_PAD, DEG_W), jnp.float32),
        mesh=mesh,
        scratch_types=[
            pltpu.VMEM((nchunks, CHUNK), jnp.int32),
            pltpu.VMEM((CHUNK, DEG_W), jnp.float32),
            pltpu.VMEM((CHUNK, DEG_W), jnp.float32),
            pltpu.VMEM_SHARED((N_PAD, DEG_W), jnp.float32),
        ],
    )
    def k(dst_hbm, out_hbm, dst_v, ones_v, buf_v, cnt_sh):
        c = lax.axis_index("c")
        s = lax.axis_index("s")
        wid = s * NC + c
        base = s * ROWS_PER_TILE

        _fill_ones_rows(ones_v, CHUNK, DEG_W)
        _zero_vmem_rows(buf_v, CHUNK, DEG_W)
        for b in range(ROWS_PER_TILE // CHUNK):
            pltpu.sync_copy(buf_v, cnt_sh.at[pl.ds(base + b * CHUNK, CHUNK)])
        plsc.subcore_barrier()

        pltpu.sync_copy(dst_hbm.at[wid], dst_v)

        def body(j, carry):
            pltpu.sync_copy(ones_v, cnt_sh.at[dst_v.at[j]], add=True)
            return carry

        lax.fori_loop(0, nchunks, body, 0)
        plsc.subcore_barrier()

        out_c = out_hbm.at[c]
        for b in range(ROWS_PER_TILE // CHUNK):
            r = base + b * CHUNK
            pltpu.sync_copy(cnt_sh.at[pl.ds(r, CHUNK)], buf_v)
            pltpu.sync_copy(buf_v, out_c.at[pl.ds(r, CHUNK)])

    return k(dst_idx)


def _sc_edge_scatter(g, src_idx, dst_idx):
    nchunks = src_idx.shape[1]
    mesh = plsc.VectorSubcoreMesh(core_axis_name="c", subcore_axis_name="s")

    @functools.partial(
        pl.kernel,
        out_type=jax.ShapeDtypeStruct((NC, N_PAD, D), jnp.float32),
        mesh=mesh,
        scratch_types=[
            pltpu.VMEM((nchunks, CHUNK), jnp.int32),
            pltpu.VMEM((nchunks, CHUNK), jnp.int32),
            pltpu.VMEM((CHUNK, D), jnp.float32),
            pltpu.VMEM_SHARED((N_PAD, D), jnp.float32),
            pltpu.SemaphoreType.DMA,
        ],
    )
    def k(g_hbm, src_hbm, dst_hbm, out_hbm, src_v, dst_v, rows_v, acc_sh, sem):
        c = lax.axis_index("c")
        s = lax.axis_index("s")
        wid = s * NC + c
        base = s * ROWS_PER_TILE

        _zero_vmem_rows(rows_v, CHUNK, D)
        for b in range(ROWS_PER_TILE // CHUNK):
            pltpu.sync_copy(rows_v, acc_sh.at[pl.ds(base + b * CHUNK, CHUNK)])
        plsc.subcore_barrier()

        pltpu.sync_copy(src_hbm.at[wid], src_v)
        pltpu.sync_copy(dst_hbm.at[wid], dst_v)

        def body(j, carry):
            pltpu.async_copy(g_hbm.at[src_v.at[j]], rows_v, sem).wait()
            pltpu.sync_copy(rows_v, acc_sh.at[dst_v.at[j]], add=True)
            return carry

        lax.fori_loop(0, nchunks, body, 0)
        plsc.subcore_barrier()

        out_c = out_hbm.at[c]
        for b in range(ROWS_PER_TILE // CHUNK):
            r = base + b * CHUNK
            pltpu.sync_copy(acc_sh.at[pl.ds(r, CHUNK)], rows_v)
            pltpu.sync_copy(rows_v, out_c.at[pl.ds(r, CHUNK)])

    return k(g, src_idx, dst_idx)


BR = 2000
NB = N_NODES // BR


def _bn_apply(y, stats, gamma, beta):
    mean = stats[0:1, :] * (1.0 / N_NODES)
    var = stats[1:2, :] * (1.0 / N_NODES) - mean * mean
    return (y - mean) * lax.rsqrt(var + EPS) * gamma + beta


def _dinv_from_deg(deg_blk):
    d = deg_blk[0, :, 0:1] + deg_blk[1, :, 0:1] + 1.0
    return lax.rsqrt(d)


def _mm(a, b):
    return jnp.dot(a, b, precision=lax.Precision.HIGHEST,
                   preferred_element_type=jnp.float32)


def _accum_stats(stats_ref, y):
    st = jnp.concatenate(
        [jnp.sum(y, axis=0, keepdims=True),
         jnp.sum(y * y, axis=0, keepdims=True),
         jnp.zeros((6, D), jnp.float32)], axis=0)
    i = pl.program_id(0)

    @pl.when(i == 0)
    def _():
        stats_ref[...] = st

    @pl.when(i != 0)
    def _():
        stats_ref[...] += st


_row_spec = pl.BlockSpec((BR, D), lambda i: (i, 0))
_full_spec = pl.BlockSpec((1, D), lambda i: (0, 0))
_stats_spec = pl.BlockSpec((8, D), lambda i: (0, 0))
_deg_spec = pl.BlockSpec((2, BR, DEG_W), lambda i: (0, i, 0))
_acc_spec = pl.BlockSpec((2, BR, D), lambda i: (0, i, 0))
_w_spec = pl.BlockSpec((D, D), lambda i: (0, 0))

_mat = jax.ShapeDtypeStruct((N_NODES, D), jnp.float32)
_stats_t = jax.ShapeDtypeStruct((8, D), jnp.float32)


def _tc_fc_stats(x, fc_W, fc_b):
    def body(x_ref, w_ref, b_ref, y_ref, stats_ref):
        y = _mm(x_ref[...], w_ref[...]) + b_ref[...]
        y_ref[...] = y
        _accum_stats(stats_ref, y)

    return pl.pallas_call(
        body,
        grid=(NB,),
        in_specs=[_row_spec, _w_spec, _full_spec],
        out_specs=(_row_spec, _stats_spec),
        out_shape=(_mat, _stats_t),
    )(x, fc_W, fc_b)


def _tc_pre_stats(accs, g_cur, conv_b, deg2):
    def body(acc_ref, g_ref, cb_ref, deg_ref, pre_ref, stats_ref):
        dinv = _dinv_from_deg(deg_ref[...])
        acc = acc_ref[0] + acc_ref[1]
        pre = dinv * (acc + g_ref[...]) + cb_ref[...]
        pre_ref[...] = pre
        _accum_stats(stats_ref, pre)

    return pl.pallas_call(
        body,
        grid=(NB,),
        in_specs=[_acc_spec, _row_spec, _full_spec, _deg_spec],
        out_specs=(_row_spec, _stats_spec),
        out_shape=(_mat, _stats_t),
    )(accs, g_cur, conv_b, deg2)


def _tc_bn_mm(y, stats, bn_g, bn_b, next_W, deg2, h_res=None):
    def body(*refs):
        if h_res is None:
            (y_ref, st_ref, bg_ref, bb_ref, w_ref, deg_ref,
             h_ref, gn_ref) = refs
            res = 0.0
        else:
            (y_ref, st_ref, bg_ref, bb_ref, w_ref, deg_ref, res_ref,
             h_ref, gn_ref) = refs
            res = res_ref[...]
        h = jnp.maximum(
            _bn_apply(y_ref[...], st_ref[...], bg_ref[...], bb_ref[...]), 0.0
        ) + res
        dinv = _dinv_from_deg(deg_ref[...])
        h_ref[...] = h
        gn_ref[...] = _mm(h, w_ref[...]) * dinv

    in_specs = [_row_spec, _stats_spec, _full_spec, _full_spec, _w_spec,
                _deg_spec]
    args = [y, stats, bn_g, bn_b, next_W, deg2]
    if h_res is not None:
        in_specs.append(_row_spec)
        args.append(h_res)
    return pl.pallas_call(
        body,
        grid=(NB,),
        in_specs=in_specs,
        out_specs=(_row_spec, _row_spec),
        out_shape=(_mat, _mat),
    )(*args)


def _tc_bn_res(y, stats, bn_g, bn_b, h_res):
    def body(y_ref, st_ref, bg_ref, bb_ref, res_ref, out_ref):
        out_ref[...] = jnp.maximum(
            _bn_apply(y_ref[...], st_ref[...], bg_ref[...], bb_ref[...]), 0.0
        ) + res_ref[...]

    return pl.pallas_call(
        body,
        grid=(NB,),
        in_specs=[_row_spec, _stats_spec, _full_spec, _full_spec, _row_spec],
        out_specs=_row_spec,
        out_shape=_mat,
    )(y, stats, bn_g, bn_b, h_res)


def kernel(x, edge_index, fc_W, fc_b, bn0_g, bn0_b, conv_W1, conv_b1,
           bn1_g, bn1_b, conv_W2, conv_b2, bn2_g, bn2_b):
    E = edge_index.shape[1]
    nchunks = -(-E // (NW * CHUNK))
    e_pad = NW * nchunks * CHUNK

    src = edge_index[0]
    dst = edge_index[1]
    pad = e_pad - E
    src_p = jnp.concatenate([src, jnp.zeros((pad,), jnp.int32)])
    dst_p = jnp.concatenate([dst, jnp.full((pad,), N_NODES, jnp.int32)])
    src_idx = src_p.reshape(NW, nchunks, CHUNK)
    dst_idx = dst_p.reshape(NW, nchunks, CHUNK)

    deg2 = _sc_degree(dst_idx)[:, :N_NODES, :]

    fc_b2 = fc_b.reshape(1, D)
    bn0_g2, bn0_b2 = bn0_g.reshape(1, D), bn0_b.reshape(1, D)
    bn1_g2, bn1_b2 = bn1_g.reshape(1, D), bn1_b.reshape(1, D)
    bn2_g2, bn2_b2 = bn2_g.reshape(1, D), bn2_b.reshape(1, D)
    b1_2, b2_2 = conv_b1.reshape(1, D), conv_b2.reshape(1, D)

    y0, st0 = _tc_fc_stats(x, fc_W, fc_b2)
    h0, g1 = _tc_bn_mm(y0, st0, bn0_g2, bn0_b2, conv_W1, deg2)
    accs1 = _sc_edge_scatter(g1, src_idx, dst_idx)
    pre1, st1 = _tc_pre_stats(accs1, g1, b1_2, deg2)
    h1, g2 = _tc_bn_mm(pre1, st1, bn1_g2, bn1_b2, conv_W2, deg2, h_res=h0)
    accs2 = _sc_edge_scatter(g2, src_idx, dst_idx)
    pre2, st2 = _tc_pre_stats(accs2, g2, b2_2, deg2)
    return _tc_bn_res(pre2, st2, bn2_g2, bn2_b2, h1)
